# baseline (speedup 1.0000x reference)
import functools

import jax
import jax.numpy as jnp
from jax import lax
from jax.experimental import pallas as pl
from jax.experimental.pallas import tpu as pltpu
from jax.experimental.pallas import tpu_sc as plsc

L = 16
K = 80
SUPC = 25


def _tc_pre_body(h_ref, wz_ref, wzi_ref, v_ref, z_ref, zi_ref, pq_ref):
    hb = h_ref[...]
    z_ref[...] = jnp.dot(hb, wz_ref[...], preferred_element_type=jnp.float32)
    zi_ref[...] = jnp.dot(hb, wzi_ref[...], preferred_element_type=jnp.float32)
    pq_ref[...] = jnp.dot(hb, v_ref[...], preferred_element_type=jnp.float32)


def _tc_pre(h, Wz, Wzi, V, block=1000):
    n, dd = h.shape
    hh = Wz.shape[1]
    return pl.pallas_call(
        _tc_pre_body,
        grid=(n // block,),
        in_specs=[
            pl.BlockSpec((block, dd), lambda i: (i, 0)),
            pl.BlockSpec((dd, hh), lambda i: (0, 0)),
            pl.BlockSpec((dd, hh), lambda i: (0, 0)),
            pl.BlockSpec((dd, 2), lambda i: (0, 0)),
        ],
        out_specs=[
            pl.BlockSpec((block, hh), lambda i: (i, 0)),
            pl.BlockSpec((block, hh), lambda i: (i, 0)),
            pl.BlockSpec((block, 2), lambda i: (i, 0)),
        ],
        out_shape=[
            jax.ShapeDtypeStruct((n, hh), jnp.float32),
            jax.ShapeDtypeStruct((n, hh), jnp.float32),
            jax.ShapeDtypeStruct((n, 2), jnp.float32),
        ],
    )(h, Wz, Wzi, V)


def _tc_mid_body(s_ref, den_ref, m_ref, zi_ref, wz_ref, wzi_ref, v_ref,
                 z_ref, zi2_ref, pq_ref):
    mv = m_ref[...]
    mm = jnp.max(mv)
    wv = jnp.exp(mv - mm)
    w0 = wv[0, 0]
    w1 = wv[1, 0]
    num = s_ref[0] * w0 + s_ref[1] * w1
    dall = den_ref[...]
    den = (w0 * jnp.sum(dall[:16], axis=0)
           + w1 * jnp.sum(dall[16:], axis=0))
    zn = jnp.where(den > 0, num / den, 0.0)
    hb = jnp.maximum(zi_ref[...] + zn, 0.0)
    z_ref[...] = jnp.dot(hb, wz_ref[...], preferred_element_type=jnp.float32)
    zi2_ref[...] = jnp.dot(hb, wzi_ref[...],
                           preferred_element_type=jnp.float32)
    pq_ref[...] = jnp.dot(hb, v_ref[...], preferred_element_type=jnp.float32)


def _tc_mid(S2, den32, m2, zi, Wz, Wzi, V, block=1000):
    n, hh = zi.shape
    return pl.pallas_call(
        _tc_mid_body,
        grid=(n // block,),
        in_specs=[
            pl.BlockSpec((2, block, hh), lambda i: (0, i, 0)),
            pl.BlockSpec((32, block, 1), lambda i: (0, i, 0)),
            pl.BlockSpec((2, L), lambda i: (0, 0)),
            pl.BlockSpec((block, hh), lambda i: (i, 0)),
            pl.BlockSpec((hh, hh), lambda i: (0, 0)),
            pl.BlockSpec((hh, hh), lambda i: (0, 0)),
            pl.BlockSpec((hh, 2), lambda i: (0, 0)),
        ],
        out_specs=[
            pl.BlockSpec((block, hh), lambda i: (i, 0)),
            pl.BlockSpec((block, hh), lambda i: (i, 0)),
            pl.BlockSpec((block, 2), lambda i: (i, 0)),
        ],
        out_shape=[
            jax.ShapeDtypeStruct((n, hh), jnp.float32),
            jax.ShapeDtypeStruct((n, hh), jnp.float32),
            jax.ShapeDtypeStruct((n, 2), jnp.float32),
        ],
    )(S2, den32, m2, zi, Wz, Wzi, V)


def _tc_post_body(s_ref, den_ref, m_ref, zi_ref, o_ref):
    mv = m_ref[...]
    mm = jnp.max(mv)
    wv = jnp.exp(mv - mm)
    w0 = wv[0, 0]
    w1 = wv[1, 0]
    num = s_ref[0] * w0 + s_ref[1] * w1
    dall = den_ref[...]
    den = (w0 * jnp.sum(dall[:16], axis=0)
           + w1 * jnp.sum(dall[16:], axis=0))
    zn = jnp.where(den > 0, num / den, 0.0)
    o_ref[...] = jnp.maximum(zi_ref[...] + zn, 0.0)


def _tc_post(S2, den32, m2, zi, block=1000):
    n, hh = zi.shape
    return pl.pallas_call(
        _tc_post_body,
        grid=(n // block,),
        in_specs=[
            pl.BlockSpec((2, block, hh), lambda i: (0, i, 0)),
            pl.BlockSpec((32, block, 1), lambda i: (0, i, 0)),
            pl.BlockSpec((2, L), lambda i: (0, 0)),
            pl.BlockSpec((block, hh), lambda i: (i, 0)),
        ],
        out_specs=pl.BlockSpec((block, hh), lambda i: (i, 0)),
        out_shape=jax.ShapeDtypeStruct((n, hh), jnp.float32),
    )(S2, den32, m2, zi)


def _make_sc_edge(n, e, hh):
    info = plsc.get_sparse_core_info()
    nc, ns = info.num_cores, info.num_subcores
    nw = nc * ns
    ep = e // nw
    nck = ep // K
    nsup = nck // SUPC
    nch = n // K
    tch = (nch + ns - 1) // ns
    mesh = plsc.VectorSubcoreMesh(core_axis_name="c", subcore_axis_name="s")

    @functools.partial(
        pl.kernel,
        out_type=[
            jax.ShapeDtypeStruct((nc, n, hh), jnp.float32),
            jax.ShapeDtypeStruct((nw * n,), jnp.float32),
            jax.ShapeDtypeStruct((nc * L,), jnp.float32),
        ],
        mesh=mesh,
        compiler_params=pltpu.CompilerParams(needs_layout_passes=False),
        scratch_types=[
            pltpu.VMEM((SUPC * K,), jnp.int32),
            pltpu.VMEM((SUPC * K,), jnp.int32),
            pltpu.VMEM((SUPC * K,), jnp.float32),
            pltpu.VMEM((K,), jnp.int32),
            pltpu.VMEM((K,), jnp.int32),
            pltpu.VMEM((n,), jnp.int32),
            pltpu.VMEM((n,), jnp.float32),
            pltpu.VMEM((K, hh), jnp.float32),
            pltpu.VMEM((K, hh), jnp.float32),
            pltpu.VMEM((L,), jnp.float32),
            pltpu.VMEM_SHARED((n, hh), jnp.float32),
            pltpu.VMEM_SHARED((ns * L,), jnp.float32),
            pltpu.SemaphoreType.DMA,
            pltpu.SemaphoreType.DMA,
            pltpu.SemaphoreType.DMA,
            pltpu.SemaphoreType.DMA,
            pltpu.SemaphoreType.DMA,
        ],
    )
    def sc_edge(z_hbm, pq_hbm, d_hbm, src_hbm, dst_hbm, c_hbm,
                s_out, den_out, m_out,
                src_v, dst_v, d_v, db0, db1, pq_v, den_v, zr0, zr1, c_v,
                s_sh, max_sh, sg0, sg1, ss0, ss1, szf):
        cid = lax.axis_index("c")
        sid = lax.axis_index("s")
        wid = cid * ns + sid
        pltpu.sync_copy(pq_hbm, pq_v)
        pltpu.sync_copy(c_hbm, c_v)
        cvec = c_v[...]
        himask = jnp.full((L,), -65536, jnp.int32)

        def stage_edges(ss):
            base = wid * ep + ss * SUPC * K
            pltpu.sync_copy(src_hbm.at[pl.ds(base, SUPC * K)], src_v)
            pltpu.sync_copy(dst_hbm.at[pl.ds(base, SUPC * K)], dst_v)
            pltpu.sync_copy(d_hbm.at[pl.ds(base, SUPC * K)], d_v)

        def escore(i):
            sl = pl.ds(i * L, L)
            sv = src_v[sl]
            dv = dst_v[sl]
            dd = d_v[sl]
            ws = plsc.load_gather(pq_v, [sv])
            wd = plsc.load_gather(pq_v, [dv])
            p = plsc.bitcast(ws & himask, jnp.float32)
            q = plsc.bitcast(wd << 16, jnp.float32)
            a = p + q + cvec * dd
            return dv, jnp.maximum(a, 0.01 * a)

        def zf(i, _):
            def zrow(j, _):
                zr0[i, pl.ds(j * L, L)] = jnp.zeros((L,), jnp.float32)
                return 0
            lax.fori_loop(0, hh // L, zrow, 0)
            return 0
        lax.fori_loop(0, K, zf, 0)

        def zout(t, _):
            ch = t * ns + sid

            @pl.when(ch < nch)
            def _():
                pltpu.async_copy(zr0, s_sh.at[pl.ds(ch * K, K)], szf)
            return 0
        lax.fori_loop(0, tch, zout, 0)

        def p1s(ss, mxs):
            stage_edges(ss)

            def p1c(i, mxr):
                _, ev = escore(i)
                return jnp.maximum(mxr, ev)
            return lax.fori_loop(0, SUPC * K // L, p1c, mxs)
        mx = lax.fori_loop(0, nsup, p1s,
                           jnp.full((L,), -jnp.inf, jnp.float32))

        def dz(i, _):
            den_v[pl.ds(i * L, L)] = jnp.zeros((L,), jnp.float32)
            return 0
        lax.fori_loop(0, n // L, dz, 0)

        def zdrain(t, _):
            ch = t * ns + sid

            @pl.when(ch < nch)
            def _():
                pltpu.make_async_copy(
                    zr0, s_sh.at[pl.ds(ch * K, K)], szf).wait()
            return 0
        lax.fori_loop(0, tch, zdrain, 0)

        zr1[0, pl.ds(0, L)] = mx
        pltpu.sync_copy(zr1.at[0, pl.ds(0, L)],
                        max_sh.at[pl.ds(sid * L, L)])
        plsc.subcore_barrier()

        def rmax(i, acc):
            pltpu.sync_copy(max_sh.at[pl.ds(i * L, L)],
                            zr1.at[0, pl.ds(0, L)])
            return jnp.maximum(acc, zr1[0, pl.ds(0, L)])
        mxv = lax.fori_loop(0, ns, rmax,
                            jnp.full((L,), -jnp.inf, jnp.float32))
        m = jnp.max(mxv)

        def g_issue(j, zr, sg):
            pltpu.async_copy(
                z_hbm.at[src_v.at[pl.ds(j * K, K)]], zr, sg)

        def g_wait(zr, sg):
            pltpu.make_async_copy(
                z_hbm.at[src_v.at[pl.ds(0, K)]], zr, sg).wait()

        def s_issue(j, zr, db, sem):
            def cpy(u, _):
                db[pl.ds(u * L, L)] = dst_v[pl.ds(j * K + u * L, L)]
                return 0
            lax.fori_loop(0, K // L, cpy, 0)
            pltpu.async_copy(zr, s_sh.at[db], sem, add=True)

        def s_wait(zr, db, sem):
            pltpu.make_async_copy(zr, s_sh.at[db], sem).wait()

        def compute(j, zr):
            def grp(u, _):
                dv, ev = escore(j * (K // L) + u)
                ee16 = jnp.exp(ev - m)
                plsc.addupdate_scatter(den_v, [dv], ee16)
                for r16 in range(L):
                    row = u * L + r16
                    sv16 = jnp.full((L,), ee16[r16], jnp.float32)
                    for cc in range(hh // L):
                        zr[row, pl.ds(cc * L, L)] = (
                            zr[row, pl.ds(cc * L, L)] * sv16)
                return 0
            lax.fori_loop(0, K // L, grp, 0)

        def p2s(ss, _):
            stage_edges(ss)
            g_issue(0, zr0, sg0)
            g_issue(1, zr1, sg1)
            g_wait(zr0, sg0)
            compute(0, zr0)
            s_issue(0, zr0, db0, ss0)

            def pair(t, _):
                ja = 2 * t + 1
                jb = 2 * t + 2
                s_wait(zr0, db0, ss0)
                g_issue(jb, zr0, sg0)
                g_wait(zr1, sg1)
                compute(ja, zr1)
                s_issue(ja, zr1, db1, ss1)

                @pl.when(jb + 1 < SUPC)
                def _():
                    s_wait(zr1, db1, ss1)
                    g_issue(jb + 1, zr1, sg1)
                g_wait(zr0, sg0)
                compute(jb, zr0)
                s_issue(jb, zr0, db0, ss0)
                return 0
            lax.fori_loop(0, (SUPC - 1) // 2, pair, 0)
            s_wait(zr0, db0, ss0)
            s_wait(zr1, db1, ss1)
            return 0
        lax.fori_loop(0, nsup, p2s, 0)
        plsc.subcore_barrier()

        def cout(t, _):
            ch = t * ns + sid

            @pl.when(ch < nch)
            def _():
                pltpu.async_copy(s_sh.at[pl.ds(ch * K, K)],
                                 s_out.at[cid, pl.ds(ch * K, K)], szf)
            return 0
        lax.fori_loop(0, tch, cout, 0)
        pltpu.sync_copy(den_v, den_out.at[pl.ds(wid * n, n)])

        def cdrain(t, _):
            ch = t * ns + sid

            @pl.when(ch < nch)
            def _():
                pltpu.make_async_copy(
                    s_sh.at[pl.ds(ch * K, K)],
                    s_out.at[cid, pl.ds(ch * K, K)], szf).wait()
            return 0
        lax.fori_loop(0, tch, cdrain, 0)

        @pl.when(sid == 0)
        def _():
            zr0[0, pl.ds(0, L)] = jnp.full((L,), m, jnp.float32)
            pltpu.sync_copy(zr0.at[0, pl.ds(0, L)],
                            m_out.at[pl.ds(cid * L, L)])

    return sc_edge


def _prep_weights(W0, W1, W2, Wa):
    hh = W1.shape[0]
    wa1 = Wa[0, :hh]
    wa2 = Wa[0, hh:2 * hh]
    c = W0[0, 0] * Wa[0, 2 * hh]
    V = jnp.stack([W1.T @ wa1, W1.T @ wa2], axis=1)
    c16 = jnp.full((L,), c, jnp.float32)
    return W1.T, W2.T, V, c16


def _pack_pq(pq):
    pu = jax.lax.bitcast_convert_type(
        pq[:, 0].astype(jnp.bfloat16), jnp.uint16).astype(jnp.uint32)
    qu = jax.lax.bitcast_convert_type(
        pq[:, 1].astype(jnp.bfloat16), jnp.uint16).astype(jnp.uint32)
    return jax.lax.bitcast_convert_type((pu << 16) | qu, jnp.int32)


def kernel(attr, d, edge_index, W0_0, W1_0, W2_0, Wa_0,
           W0_1, W1_1, W2_1, Wa_1):
    n, _ = attr.shape
    e = edge_index.shape[1]
    hh = W1_0.shape[0]
    src_r = edge_index[0]
    dst_r = edge_index[1]
    d1 = d[:, 0]
    sc_edge = _make_sc_edge(n, e, hh)
    Wz0, Wzi0, V0, c0 = _prep_weights(W0_0, W1_0, W2_0, Wa_0)
    Wz1, Wzi1, V1, c1 = _prep_weights(W0_1, W1_1, W2_1, Wa_1)
    z0, zi0, pq0 = _tc_pre(attr, Wz0, Wzi0, V0)
    S0, den0, m0 = sc_edge(z0, _pack_pq(pq0), d1, src_r, dst_r, c0)
    z1, zi1, pq1 = _tc_mid(S0, den0.reshape(32, n, 1), m0.reshape(2, L),
                           zi0, Wz1, Wzi1, V1)
    S1, den1, m1 = sc_edge(z1, _pack_pq(pq1), d1, src_r, dst_r, c1)
    return _tc_post(S1, den1.reshape(32, n, 1), m1.reshape(2, L), zi1)


# speedup vs baseline: 1.7972x; 1.7972x over previous
import functools

import jax
import jax.numpy as jnp
from jax import lax
from jax.experimental import pallas as pl
from jax.experimental.pallas import tpu as pltpu
from jax.experimental.pallas import tpu_sc as plsc

L = 16
K = 80
SUPC = 25


def _tc_pre_body(h_ref, wz_ref, wzi_ref, v_ref, z_ref, zi_ref, pq_ref):
    hb = h_ref[...]
    z_ref[...] = jnp.dot(hb, wz_ref[...], preferred_element_type=jnp.float32)
    zi_ref[...] = jnp.dot(hb, wzi_ref[...], preferred_element_type=jnp.float32)
    pq_ref[...] = jnp.dot(hb, v_ref[...], preferred_element_type=jnp.float32)


def _tc_pre(h, Wz, Wzi, V, block=1000):
    n, dd = h.shape
    hh = Wz.shape[1]
    return pl.pallas_call(
        _tc_pre_body,
        grid=(n // block,),
        in_specs=[
            pl.BlockSpec((block, dd), lambda i: (i, 0)),
            pl.BlockSpec((dd, hh), lambda i: (0, 0)),
            pl.BlockSpec((dd, hh), lambda i: (0, 0)),
            pl.BlockSpec((dd, 2), lambda i: (0, 0)),
        ],
        out_specs=[
            pl.BlockSpec((block, hh), lambda i: (i, 0)),
            pl.BlockSpec((block, hh), lambda i: (i, 0)),
            pl.BlockSpec((block, 2), lambda i: (i, 0)),
        ],
        out_shape=[
            jax.ShapeDtypeStruct((n, hh), jnp.float32),
            jax.ShapeDtypeStruct((n, hh), jnp.float32),
            jax.ShapeDtypeStruct((n, 2), jnp.float32),
        ],
    )(h, Wz, Wzi, V)


def _tc_mid_body(s_ref, den_ref, m_ref, zi_ref, wz_ref, wzi_ref, v_ref,
                 z_ref, zi2_ref, pq_ref):
    mv = m_ref[...]
    mm = jnp.max(mv)
    wv = jnp.exp(mv - mm)
    w0 = wv[0, 0]
    w1 = wv[1, 0]
    num = s_ref[0] * w0 + s_ref[1] * w1
    dall = den_ref[...]
    den = (w0 * jnp.sum(dall[:, :16], axis=1)
           + w1 * jnp.sum(dall[:, 16:], axis=1))[:, None]
    zn = jnp.where(den > 0, num / den, 0.0)
    hb = jnp.maximum(zi_ref[...] + zn, 0.0)
    z_ref[...] = jnp.dot(hb, wz_ref[...], preferred_element_type=jnp.float32)
    zi2_ref[...] = jnp.dot(hb, wzi_ref[...],
                           preferred_element_type=jnp.float32)
    pq_ref[...] = jnp.dot(hb, v_ref[...], preferred_element_type=jnp.float32)


def _tc_mid(S2, den32, m2, zi, Wz, Wzi, V, block=1000):
    n, hh = zi.shape
    return pl.pallas_call(
        _tc_mid_body,
        grid=(n // block,),
        in_specs=[
            pl.BlockSpec((2, block, hh), lambda i: (0, i, 0)),
            pl.BlockSpec((block, 32), lambda i: (i, 0)),
            pl.BlockSpec((2, L), lambda i: (0, 0)),
            pl.BlockSpec((block, hh), lambda i: (i, 0)),
            pl.BlockSpec((hh, hh), lambda i: (0, 0)),
            pl.BlockSpec((hh, hh), lambda i: (0, 0)),
            pl.BlockSpec((hh, 2), lambda i: (0, 0)),
        ],
        out_specs=[
            pl.BlockSpec((block, hh), lambda i: (i, 0)),
            pl.BlockSpec((block, hh), lambda i: (i, 0)),
            pl.BlockSpec((block, 2), lambda i: (i, 0)),
        ],
        out_shape=[
            jax.ShapeDtypeStruct((n, hh), jnp.float32),
            jax.ShapeDtypeStruct((n, hh), jnp.float32),
            jax.ShapeDtypeStruct((n, 2), jnp.float32),
        ],
    )(S2, den32, m2, zi, Wz, Wzi, V)


def _tc_post_body(s_ref, den_ref, m_ref, zi_ref, o_ref):
    mv = m_ref[...]
    mm = jnp.max(mv)
    wv = jnp.exp(mv - mm)
    w0 = wv[0, 0]
    w1 = wv[1, 0]
    num = s_ref[0] * w0 + s_ref[1] * w1
    dall = den_ref[...]
    den = (w0 * jnp.sum(dall[:, :16], axis=1)
           + w1 * jnp.sum(dall[:, 16:], axis=1))[:, None]
    zn = jnp.where(den > 0, num / den, 0.0)
    o_ref[...] = jnp.maximum(zi_ref[...] + zn, 0.0)


def _tc_post(S2, den32, m2, zi, block=1000):
    n, hh = zi.shape
    return pl.pallas_call(
        _tc_post_body,
        grid=(n // block,),
        in_specs=[
            pl.BlockSpec((2, block, hh), lambda i: (0, i, 0)),
            pl.BlockSpec((block, 32), lambda i: (i, 0)),
            pl.BlockSpec((2, L), lambda i: (0, 0)),
            pl.BlockSpec((block, hh), lambda i: (i, 0)),
        ],
        out_specs=pl.BlockSpec((block, hh), lambda i: (i, 0)),
        out_shape=jax.ShapeDtypeStruct((n, hh), jnp.float32),
    )(S2, den32, m2, zi)


def _make_sc_edge(n, e, hh):
    info = plsc.get_sparse_core_info()
    nc, ns = info.num_cores, info.num_subcores
    nw = nc * ns
    ep = e // nw
    nck = ep // K
    nsup = nck // SUPC
    nch = n // K
    tch = (nch + ns - 1) // ns
    mesh = plsc.VectorSubcoreMesh(core_axis_name="c", subcore_axis_name="s")

    @functools.partial(
        pl.kernel,
        out_type=[
            jax.ShapeDtypeStruct((nc, n, hh), jnp.float32),
            jax.ShapeDtypeStruct((nw * n,), jnp.float32),
            jax.ShapeDtypeStruct((nc * L,), jnp.float32),
        ],
        mesh=mesh,
        compiler_params=pltpu.CompilerParams(needs_layout_passes=False),
        scratch_types=[
            pltpu.VMEM((SUPC * K,), jnp.int32),
            pltpu.VMEM((SUPC * K,), jnp.int32),
            pltpu.VMEM((SUPC * K,), jnp.float32),
            pltpu.VMEM((K,), jnp.int32),
            pltpu.VMEM((K,), jnp.int32),
            pltpu.VMEM((n,), jnp.int32),
            pltpu.VMEM((n,), jnp.float32),
            pltpu.VMEM((K, hh), jnp.float32),
            pltpu.VMEM((K, hh), jnp.float32),
            pltpu.VMEM((L,), jnp.float32),
            pltpu.VMEM_SHARED((n, hh), jnp.float32),
            pltpu.VMEM_SHARED((ns * L,), jnp.float32),
            pltpu.SemaphoreType.DMA,
            pltpu.SemaphoreType.DMA,
            pltpu.SemaphoreType.DMA,
            pltpu.SemaphoreType.DMA,
            pltpu.SemaphoreType.DMA,
        ],
    )
    def sc_edge(z_hbm, pq_hbm, d_hbm, src_hbm, dst_hbm, c_hbm,
                s_out, den_out, m_out,
                src_v, dst_v, d_v, db0, db1, pq_v, den_v, zr0, zr1, c_v,
                s_sh, max_sh, sg0, sg1, ss0, ss1, szf):
        cid = lax.axis_index("c")
        sid = lax.axis_index("s")
        wid = cid * ns + sid
        pltpu.sync_copy(pq_hbm, pq_v)
        pltpu.sync_copy(c_hbm, c_v)
        cvec = c_v[...]
        himask = jnp.full((L,), -65536, jnp.int32)

        def stage_edges(ss):
            base = wid * ep + ss * SUPC * K
            pltpu.sync_copy(src_hbm.at[pl.ds(base, SUPC * K)], src_v)
            pltpu.sync_copy(dst_hbm.at[pl.ds(base, SUPC * K)], dst_v)
            pltpu.sync_copy(d_hbm.at[pl.ds(base, SUPC * K)], d_v)

        def escore(i):
            sl = pl.ds(i * L, L)
            sv = src_v[sl]
            dv = dst_v[sl]
            dd = d_v[sl]
            ws = plsc.load_gather(pq_v, [sv])
            wd = plsc.load_gather(pq_v, [dv])
            p = plsc.bitcast(ws & himask, jnp.float32)
            q = plsc.bitcast(wd << 16, jnp.float32)
            a = p + q + cvec * dd
            return dv, jnp.maximum(a, 0.01 * a)

        def zf(i, _):
            def zrow(j, _):
                zr0[i, pl.ds(j * L, L)] = jnp.zeros((L,), jnp.float32)
                return 0
            lax.fori_loop(0, hh // L, zrow, 0)
            return 0
        lax.fori_loop(0, K, zf, 0)

        def zout(t, _):
            ch = t * ns + sid

            @pl.when(ch < nch)
            def _():
                pltpu.async_copy(zr0, s_sh.at[pl.ds(ch * K, K)], szf)
            return 0
        lax.fori_loop(0, tch, zout, 0)

        def p1s(ss, mxs):
            stage_edges(ss)

            def p1c(i, mxr):
                _, ev = escore(i)
                return jnp.maximum(mxr, ev)
            return lax.fori_loop(0, SUPC * K // L, p1c, mxs)
        mx = lax.fori_loop(0, nsup, p1s,
                           jnp.full((L,), -jnp.inf, jnp.float32))

        def dz(i, _):
            den_v[pl.ds(i * L, L)] = jnp.zeros((L,), jnp.float32)
            return 0
        lax.fori_loop(0, n // L, dz, 0)

        def zdrain(t, _):
            ch = t * ns + sid

            @pl.when(ch < nch)
            def _():
                pltpu.make_async_copy(
                    zr0, s_sh.at[pl.ds(ch * K, K)], szf).wait()
            return 0
        lax.fori_loop(0, tch, zdrain, 0)

        zr1[0, pl.ds(0, L)] = mx
        pltpu.sync_copy(zr1.at[0, pl.ds(0, L)],
                        max_sh.at[pl.ds(sid * L, L)])
        plsc.subcore_barrier()

        def rmax(i, acc):
            pltpu.sync_copy(max_sh.at[pl.ds(i * L, L)],
                            zr1.at[0, pl.ds(0, L)])
            return jnp.maximum(acc, zr1[0, pl.ds(0, L)])
        mxv = lax.fori_loop(0, ns, rmax,
                            jnp.full((L,), -jnp.inf, jnp.float32))
        m = jnp.max(mxv)

        def g_issue(j, zr, sg):
            pltpu.async_copy(
                z_hbm.at[src_v.at[pl.ds(j * K, K)]], zr, sg)

        def g_wait(zr, sg):
            pltpu.make_async_copy(
                z_hbm.at[src_v.at[pl.ds(0, K)]], zr, sg).wait()

        def s_issue(j, zr, db, sem):
            def cpy(u, _):
                db[pl.ds(u * L, L)] = dst_v[pl.ds(j * K + u * L, L)]
                return 0
            lax.fori_loop(0, K // L, cpy, 0)
            pltpu.async_copy(zr, s_sh.at[db], sem, add=True)

        def s_wait(zr, db, sem):
            pltpu.make_async_copy(zr, s_sh.at[db], sem).wait()

        def compute(j, zr):
            def grp(u, _):
                dv, ev = escore(j * (K // L) + u)
                ee16 = jnp.exp(ev - m)
                plsc.addupdate_scatter(den_v, [dv], ee16)
                for r16 in range(L):
                    row = u * L + r16
                    sv16 = jnp.full((L,), ee16[r16], jnp.float32)
                    for cc in range(hh // L):
                        zr[row, pl.ds(cc * L, L)] = (
                            zr[row, pl.ds(cc * L, L)] * sv16)
                return 0
            lax.fori_loop(0, K // L, grp, 0)

        def p2s(ss, _):
            stage_edges(ss)
            g_issue(0, zr0, sg0)
            g_issue(1, zr1, sg1)
            g_wait(zr0, sg0)
            compute(0, zr0)
            s_issue(0, zr0, db0, ss0)

            def pair(t, _):
                ja = 2 * t + 1
                jb = 2 * t + 2
                s_wait(zr0, db0, ss0)
                g_issue(jb, zr0, sg0)
                g_wait(zr1, sg1)
                compute(ja, zr1)
                s_issue(ja, zr1, db1, ss1)

                @pl.when(jb + 1 < SUPC)
                def _():
                    s_wait(zr1, db1, ss1)
                    g_issue(jb + 1, zr1, sg1)
                g_wait(zr0, sg0)
                compute(jb, zr0)
                s_issue(jb, zr0, db0, ss0)
                return 0
            lax.fori_loop(0, (SUPC - 1) // 2, pair, 0)
            s_wait(zr0, db0, ss0)
            s_wait(zr1, db1, ss1)
            return 0
        lax.fori_loop(0, nsup, p2s, 0)
        plsc.subcore_barrier()

        def cout(t, _):
            ch = t * ns + sid

            @pl.when(ch < nch)
            def _():
                pltpu.async_copy(s_sh.at[pl.ds(ch * K, K)],
                                 s_out.at[cid, pl.ds(ch * K, K)], szf)
            return 0
        lax.fori_loop(0, tch, cout, 0)
        pltpu.sync_copy(den_v, den_out.at[pl.ds(wid * n, n)])

        def cdrain(t, _):
            ch = t * ns + sid

            @pl.when(ch < nch)
            def _():
                pltpu.make_async_copy(
                    s_sh.at[pl.ds(ch * K, K)],
                    s_out.at[cid, pl.ds(ch * K, K)], szf).wait()
            return 0
        lax.fori_loop(0, tch, cdrain, 0)

        @pl.when(sid == 0)
        def _():
            zr0[0, pl.ds(0, L)] = jnp.full((L,), m, jnp.float32)
            pltpu.sync_copy(zr0.at[0, pl.ds(0, L)],
                            m_out.at[pl.ds(cid * L, L)])

    return sc_edge


def _prep_weights(W0, W1, W2, Wa):
    hh = W1.shape[0]
    wa1 = Wa[0, :hh]
    wa2 = Wa[0, hh:2 * hh]
    c = W0[0, 0] * Wa[0, 2 * hh]
    V = jnp.stack([W1.T @ wa1, W1.T @ wa2], axis=1)
    c16 = jnp.full((L,), c, jnp.float32)
    return W1.T, W2.T, V, c16


def _pack_pq(pq):
    pu = jax.lax.bitcast_convert_type(
        pq[:, 0].astype(jnp.bfloat16), jnp.uint16).astype(jnp.uint32)
    qu = jax.lax.bitcast_convert_type(
        pq[:, 1].astype(jnp.bfloat16), jnp.uint16).astype(jnp.uint32)
    return jax.lax.bitcast_convert_type((pu << 16) | qu, jnp.int32)


def kernel(attr, d, edge_index, W0_0, W1_0, W2_0, Wa_0,
           W0_1, W1_1, W2_1, Wa_1):
    n, _ = attr.shape
    e = edge_index.shape[1]
    hh = W1_0.shape[0]
    src_r = edge_index[0]
    dst_r = edge_index[1]
    d1 = d[:, 0]
    sc_edge = _make_sc_edge(n, e, hh)
    Wz0, Wzi0, V0, c0 = _prep_weights(W0_0, W1_0, W2_0, Wa_0)
    Wz1, Wzi1, V1, c1 = _prep_weights(W0_1, W1_1, W2_1, Wa_1)
    z0, zi0, pq0 = _tc_pre(attr, Wz0, Wzi0, V0)
    S0, den0, m0 = sc_edge(z0, _pack_pq(pq0), d1, src_r, dst_r, c0)
    z1, zi1, pq1 = _tc_mid(S0, den0.reshape(32, n).T, m0.reshape(2, L),
                           zi0, Wz1, Wzi1, V1)
    S1, den1, m1 = sc_edge(z1, _pack_pq(pq1), d1, src_r, dst_r, c1)
    return _tc_post(S1, den1.reshape(32, n).T, m1.reshape(2, L), zi1)


# speedup vs baseline: 1.8051x; 1.0044x over previous
import functools

import jax
import jax.numpy as jnp
from jax import lax
from jax.experimental import pallas as pl
from jax.experimental.pallas import tpu as pltpu
from jax.experimental.pallas import tpu_sc as plsc

L = 16
K = 80
SUPC = 25


def _pack_cols(pqf):
    pu = lax.bitcast_convert_type(pqf[:, 0:1], jnp.uint32) & jnp.uint32(
        0xFFFF0000)
    qu = lax.bitcast_convert_type(pqf[:, 1:2], jnp.uint32) >> 16
    return lax.bitcast_convert_type(pu | qu, jnp.int32)


def _tc_pre_body(h_ref, wz_ref, wzi_ref, v_ref, z_ref, zi_ref, pq_ref):
    hb = h_ref[...]
    z_ref[...] = jnp.dot(hb, wz_ref[...], preferred_element_type=jnp.float32)
    zi_ref[...] = jnp.dot(hb, wzi_ref[...], preferred_element_type=jnp.float32)
    pq_ref[...] = _pack_cols(
        jnp.dot(hb, v_ref[...], preferred_element_type=jnp.float32))


def _tc_pre(h, Wz, Wzi, V, block=1000):
    n, dd = h.shape
    hh = Wz.shape[1]
    return pl.pallas_call(
        _tc_pre_body,
        grid=(n // block,),
        in_specs=[
            pl.BlockSpec((block, dd), lambda i: (i, 0)),
            pl.BlockSpec((dd, hh), lambda i: (0, 0)),
            pl.BlockSpec((dd, hh), lambda i: (0, 0)),
            pl.BlockSpec((dd, 2), lambda i: (0, 0)),
        ],
        out_specs=[
            pl.BlockSpec((block, hh), lambda i: (i, 0)),
            pl.BlockSpec((block, hh), lambda i: (i, 0)),
            pl.BlockSpec((block, 1), lambda i: (i, 0)),
        ],
        out_shape=[
            jax.ShapeDtypeStruct((n, hh), jnp.float32),
            jax.ShapeDtypeStruct((n, hh), jnp.float32),
            jax.ShapeDtypeStruct((n, 1), jnp.int32),
        ],
    )(h, Wz, Wzi, V)


def _tc_mid_body(s_ref, den_ref, m_ref, zi_ref, wz_ref, wzi_ref, v_ref,
                 z_ref, zi2_ref, pq_ref):
    mv = m_ref[...]
    mm = jnp.max(mv)
    wv = jnp.exp(mv - mm)
    w0 = wv[0, 0]
    w1 = wv[1, 0]
    num = s_ref[0] * w0 + s_ref[1] * w1
    dall = den_ref[...]
    den = (w0 * jnp.sum(dall[:, :16], axis=1)
           + w1 * jnp.sum(dall[:, 16:], axis=1))[:, None]
    zn = jnp.where(den > 0, num / den, 0.0)
    hb = jnp.maximum(zi_ref[...] + zn, 0.0)
    z_ref[...] = jnp.dot(hb, wz_ref[...], preferred_element_type=jnp.float32)
    zi2_ref[...] = jnp.dot(hb, wzi_ref[...],
                           preferred_element_type=jnp.float32)
    pq_ref[...] = _pack_cols(
        jnp.dot(hb, v_ref[...], preferred_element_type=jnp.float32))


def _tc_mid(S2, den32, m2, zi, Wz, Wzi, V, block=1000):
    n, hh = zi.shape
    return pl.pallas_call(
        _tc_mid_body,
        grid=(n // block,),
        in_specs=[
            pl.BlockSpec((2, block, hh), lambda i: (0, i, 0)),
            pl.BlockSpec((block, 32), lambda i: (i, 0)),
            pl.BlockSpec((2, L), lambda i: (0, 0)),
            pl.BlockSpec((block, hh), lambda i: (i, 0)),
            pl.BlockSpec((hh, hh), lambda i: (0, 0)),
            pl.BlockSpec((hh, hh), lambda i: (0, 0)),
            pl.BlockSpec((hh, 2), lambda i: (0, 0)),
        ],
        out_specs=[
            pl.BlockSpec((block, hh), lambda i: (i, 0)),
            pl.BlockSpec((block, hh), lambda i: (i, 0)),
            pl.BlockSpec((block, 1), lambda i: (i, 0)),
        ],
        out_shape=[
            jax.ShapeDtypeStruct((n, hh), jnp.float32),
            jax.ShapeDtypeStruct((n, hh), jnp.float32),
            jax.ShapeDtypeStruct((n, 1), jnp.int32),
        ],
    )(S2, den32, m2, zi, Wz, Wzi, V)


def _tc_post_body(s_ref, den_ref, m_ref, zi_ref, o_ref):
    mv = m_ref[...]
    mm = jnp.max(mv)
    wv = jnp.exp(mv - mm)
    w0 = wv[0, 0]
    w1 = wv[1, 0]
    num = s_ref[0] * w0 + s_ref[1] * w1
    dall = den_ref[...]
    den = (w0 * jnp.sum(dall[:, :16], axis=1)
           + w1 * jnp.sum(dall[:, 16:], axis=1))[:, None]
    zn = jnp.where(den > 0, num / den, 0.0)
    o_ref[...] = jnp.maximum(zi_ref[...] + zn, 0.0)


def _tc_post(S2, den32, m2, zi, block=1000):
    n, hh = zi.shape
    return pl.pallas_call(
        _tc_post_body,
        grid=(n // block,),
        in_specs=[
            pl.BlockSpec((2, block, hh), lambda i: (0, i, 0)),
            pl.BlockSpec((block, 32), lambda i: (i, 0)),
            pl.BlockSpec((2, L), lambda i: (0, 0)),
            pl.BlockSpec((block, hh), lambda i: (i, 0)),
        ],
        out_specs=pl.BlockSpec((block, hh), lambda i: (i, 0)),
        out_shape=jax.ShapeDtypeStruct((n, hh), jnp.float32),
    )(S2, den32, m2, zi)


def _make_sc_edge(n, e, hh):
    info = plsc.get_sparse_core_info()
    nc, ns = info.num_cores, info.num_subcores
    nw = nc * ns
    ep = e // nw
    nck = ep // K
    nsup = nck // SUPC
    nch = n // K
    tch = (nch + ns - 1) // ns
    mesh = plsc.VectorSubcoreMesh(core_axis_name="c", subcore_axis_name="s")

    @functools.partial(
        pl.kernel,
        out_type=[
            jax.ShapeDtypeStruct((nc, n, hh), jnp.float32),
            jax.ShapeDtypeStruct((nw * n,), jnp.float32),
            jax.ShapeDtypeStruct((nc * L,), jnp.float32),
        ],
        mesh=mesh,
        compiler_params=pltpu.CompilerParams(needs_layout_passes=False),
        scratch_types=[
            pltpu.VMEM((SUPC * K,), jnp.int32),
            pltpu.VMEM((SUPC * K,), jnp.int32),
            pltpu.VMEM((SUPC * K,), jnp.float32),
            pltpu.VMEM((K,), jnp.int32),
            pltpu.VMEM((K,), jnp.int32),
            pltpu.VMEM((n,), jnp.int32),
            pltpu.VMEM((n,), jnp.float32),
            pltpu.VMEM((K, hh), jnp.float32),
            pltpu.VMEM((K, hh), jnp.float32),
            pltpu.VMEM((L,), jnp.float32),
            pltpu.VMEM_SHARED((n, hh), jnp.float32),
            pltpu.VMEM_SHARED((ns * L,), jnp.float32),
            pltpu.SemaphoreType.DMA,
            pltpu.SemaphoreType.DMA,
            pltpu.SemaphoreType.DMA,
            pltpu.SemaphoreType.DMA,
            pltpu.SemaphoreType.DMA,
        ],
    )
    def sc_edge(z_hbm, pq_hbm, d_hbm, src_hbm, dst_hbm, c_hbm,
                s_out, den_out, m_out,
                src_v, dst_v, d_v, db0, db1, pq_v, den_v, zr0, zr1, c_v,
                s_sh, max_sh, sg0, sg1, ss0, ss1, szf):
        cid = lax.axis_index("c")
        sid = lax.axis_index("s")
        wid = cid * ns + sid
        pltpu.sync_copy(pq_hbm, pq_v)
        pltpu.sync_copy(c_hbm, c_v)
        cvec = c_v[...]
        himask = jnp.full((L,), -65536, jnp.int32)

        def stage_edges(ss):
            base = wid * ep + ss * SUPC * K
            pltpu.sync_copy(src_hbm.at[pl.ds(base, SUPC * K)], src_v)
            pltpu.sync_copy(dst_hbm.at[pl.ds(base, SUPC * K)], dst_v)
            pltpu.sync_copy(d_hbm.at[pl.ds(base, SUPC * K)], d_v)

        def escore(i):
            sl = pl.ds(i * L, L)
            sv = src_v[sl]
            dv = dst_v[sl]
            dd = d_v[sl]
            ws = plsc.load_gather(pq_v, [sv])
            wd = plsc.load_gather(pq_v, [dv])
            p = plsc.bitcast(ws & himask, jnp.float32)
            q = plsc.bitcast(wd << 16, jnp.float32)
            a = p + q + cvec * dd
            return dv, jnp.maximum(a, 0.01 * a)

        def zf(i, _):
            def zrow(j, _):
                zr0[i, pl.ds(j * L, L)] = jnp.zeros((L,), jnp.float32)
                return 0
            lax.fori_loop(0, hh // L, zrow, 0)
            return 0
        lax.fori_loop(0, K, zf, 0)

        def zout(t, _):
            ch = t * ns + sid

            @pl.when(ch < nch)
            def _():
                pltpu.async_copy(zr0, s_sh.at[pl.ds(ch * K, K)], szf)
            return 0
        lax.fori_loop(0, tch, zout, 0)

        def p1s(ss, mxs):
            stage_edges(ss)

            def p1c(i, mxr):
                _, ev = escore(i)
                return jnp.maximum(mxr, ev)
            return lax.fori_loop(0, SUPC * K // L, p1c, mxs)
        mx = lax.fori_loop(0, nsup, p1s,
                           jnp.full((L,), -jnp.inf, jnp.float32))

        def dz(i, _):
            den_v[pl.ds(i * L, L)] = jnp.zeros((L,), jnp.float32)
            return 0
        lax.fori_loop(0, n // L, dz, 0)

        def zdrain(t, _):
            ch = t * ns + sid

            @pl.when(ch < nch)
            def _():
                pltpu.make_async_copy(
                    zr0, s_sh.at[pl.ds(ch * K, K)], szf).wait()
            return 0
        lax.fori_loop(0, tch, zdrain, 0)

        zr1[0, pl.ds(0, L)] = mx
        pltpu.sync_copy(zr1.at[0, pl.ds(0, L)],
                        max_sh.at[pl.ds(sid * L, L)])
        plsc.subcore_barrier()

        def rmax(i, acc):
            pltpu.sync_copy(max_sh.at[pl.ds(i * L, L)],
                            zr1.at[0, pl.ds(0, L)])
            return jnp.maximum(acc, zr1[0, pl.ds(0, L)])
        mxv = lax.fori_loop(0, ns, rmax,
                            jnp.full((L,), -jnp.inf, jnp.float32))
        m = jnp.max(mxv)

        def g_issue(j, zr, sg):
            pltpu.async_copy(
                z_hbm.at[src_v.at[pl.ds(j * K, K)]], zr, sg)

        def g_wait(zr, sg):
            pltpu.make_async_copy(
                z_hbm.at[src_v.at[pl.ds(0, K)]], zr, sg).wait()

        def s_issue(j, zr, db, sem):
            def cpy(u, _):
                db[pl.ds(u * L, L)] = dst_v[pl.ds(j * K + u * L, L)]
                return 0
            lax.fori_loop(0, K // L, cpy, 0)
            pltpu.async_copy(zr, s_sh.at[db], sem, add=True)

        def s_wait(zr, db, sem):
            pltpu.make_async_copy(zr, s_sh.at[db], sem).wait()

        def compute(j, zr):
            def grp(u, _):
                dv, ev = escore(j * (K // L) + u)
                ee16 = jnp.exp(ev - m)
                plsc.addupdate_scatter(den_v, [dv], ee16)
                for r16 in range(L):
                    row = u * L + r16
                    sv16 = jnp.full((L,), ee16[r16], jnp.float32)
                    for cc in range(hh // L):
                        zr[row, pl.ds(cc * L, L)] = (
                            zr[row, pl.ds(cc * L, L)] * sv16)
                return 0
            lax.fori_loop(0, K // L, grp, 0)

        def p2s(ss, _):
            stage_edges(ss)
            g_issue(0, zr0, sg0)
            g_issue(1, zr1, sg1)
            g_wait(zr0, sg0)
            compute(0, zr0)
            s_issue(0, zr0, db0, ss0)

            def pair(t, _):
                ja = 2 * t + 1
                jb = 2 * t + 2
                s_wait(zr0, db0, ss0)
                g_issue(jb, zr0, sg0)
                g_wait(zr1, sg1)
                compute(ja, zr1)
                s_issue(ja, zr1, db1, ss1)

                @pl.when(jb + 1 < SUPC)
                def _():
                    s_wait(zr1, db1, ss1)
                    g_issue(jb + 1, zr1, sg1)
                g_wait(zr0, sg0)
                compute(jb, zr0)
                s_issue(jb, zr0, db0, ss0)
                return 0
            lax.fori_loop(0, (SUPC - 1) // 2, pair, 0)
            s_wait(zr0, db0, ss0)
            s_wait(zr1, db1, ss1)
            return 0
        lax.fori_loop(0, nsup, p2s, 0)
        plsc.subcore_barrier()

        def cout(t, _):
            ch = t * ns + sid

            @pl.when(ch < nch)
            def _():
                pltpu.async_copy(s_sh.at[pl.ds(ch * K, K)],
                                 s_out.at[cid, pl.ds(ch * K, K)], szf)
            return 0
        lax.fori_loop(0, tch, cout, 0)
        pltpu.sync_copy(den_v, den_out.at[pl.ds(wid * n, n)])

        def cdrain(t, _):
            ch = t * ns + sid

            @pl.when(ch < nch)
            def _():
                pltpu.make_async_copy(
                    s_sh.at[pl.ds(ch * K, K)],
                    s_out.at[cid, pl.ds(ch * K, K)], szf).wait()
            return 0
        lax.fori_loop(0, tch, cdrain, 0)

        @pl.when(sid == 0)
        def _():
            zr0[0, pl.ds(0, L)] = jnp.full((L,), m, jnp.float32)
            pltpu.sync_copy(zr0.at[0, pl.ds(0, L)],
                            m_out.at[pl.ds(cid * L, L)])

    return sc_edge


def _prep_weights(W0, W1, W2, Wa):
    hh = W1.shape[0]
    wa1 = Wa[0, :hh]
    wa2 = Wa[0, hh:2 * hh]
    c = W0[0, 0] * Wa[0, 2 * hh]
    V = jnp.stack([W1.T @ wa1, W1.T @ wa2], axis=1)
    c16 = jnp.full((L,), c, jnp.float32)
    return W1.T, W2.T, V, c16


def kernel(attr, d, edge_index, W0_0, W1_0, W2_0, Wa_0,
           W0_1, W1_1, W2_1, Wa_1):
    n, _ = attr.shape
    e = edge_index.shape[1]
    hh = W1_0.shape[0]
    src_r = edge_index[0]
    dst_r = edge_index[1]
    d1 = d[:, 0]
    sc_edge = _make_sc_edge(n, e, hh)
    Wz0, Wzi0, V0, c0 = _prep_weights(W0_0, W1_0, W2_0, Wa_0)
    Wz1, Wzi1, V1, c1 = _prep_weights(W0_1, W1_1, W2_1, Wa_1)
    z0, zi0, pq0 = _tc_pre(attr, Wz0, Wzi0, V0)
    S0, den0, m0 = sc_edge(z0, pq0.reshape(n), d1, src_r, dst_r, c0)
    z1, zi1, pq1 = _tc_mid(S0, den0.reshape(32, n).T, m0.reshape(2, L),
                           zi0, Wz1, Wzi1, V1)
    S1, den1, m1 = sc_edge(z1, pq1.reshape(n), d1, src_r, dst_r, c1)
    return _tc_post(S1, den1.reshape(32, n).T, m1.reshape(2, L), zi1)
